# Initial kernel scaffold; baseline (speedup 1.0000x reference)
import jax
import jax.numpy as jnp
from jax.experimental import pallas as pl


def kernel(x, edge_index, edge_attr, W1, root1, b1, W2, root2, b2):
    raise NotImplementedError("write your pallas kernel here")


# speedup vs baseline: 6.6636x; 6.6636x over previous
import functools

import jax
import jax.numpy as jnp
from jax import lax
from jax.experimental import pallas as pl
from jax.experimental.pallas import tpu as pltpu
from jax.experimental.pallas import tpu_sc as plsc

N = 10000
E = 320000
HID = 16

NC = 2
NS = 16
NW = NC * NS
CHUNK = 128
NCH = -(-(E // NW) // CHUNK)
EPW = NCH * CHUNK
EPAD = EPW * NW
ZR = 632
NROW = ZR * NS

_mesh = plsc.VectorSubcoreMesh(core_axis_name="c", subcore_axis_name="s")


def _make_edge_pass(out_w):
    with_count = out_w == 32

    @functools.partial(
        pl.kernel,
        out_type=jax.ShapeDtypeStruct((NC, NROW, out_w), jnp.float32),
        mesh=_mesh,
        compiler_params=pltpu.CompilerParams(use_tc_tiling_on_sc=False),
        scratch_types=[
            pltpu.VMEM_SHARED((NROW, out_w), jnp.float32),
            pltpu.VMEM((CHUNK,), jnp.int32),
            pltpu.VMEM((CHUNK,), jnp.int32),
            pltpu.VMEM((CHUNK,), jnp.float32),
            pltpu.VMEM((CHUNK, 32), jnp.float32),
            pltpu.VMEM((CHUNK, out_w), jnp.float32),
            pltpu.VMEM((ZR, out_w), jnp.float32),
            pltpu.SemaphoreType.DMA,
        ],
    )
    def edge_pass(table, srcr, dstr, ur, out, acc, src_v, dst_v, u_v,
                  rows_v, msg_v, zbuf, sem):
        c = lax.axis_index("c")
        s = lax.axis_index("s")
        wid = c * NS + s
        zero16 = jnp.zeros((16,), jnp.float32)

        def zrow(i, carry):
            zbuf[i, 0:16] = zero16
            if with_count:
                zbuf[i, 16:32] = zero16
            return carry
        lax.fori_loop(0, ZR, zrow, 0)
        pltpu.sync_copy(zbuf, acc.at[pl.ds(s * ZR, ZR)])

        if with_count:
            cvec = jnp.where(lax.iota(jnp.int32, 16) == 0,
                             jnp.float32(1.0), jnp.float32(0.0))

            def mrow(i, carry):
                msg_v[i, 16:32] = cvec
                return carry
            lax.fori_loop(0, CHUNK, mrow, 0)

        plsc.subcore_barrier()

        base = wid * EPW

        def chunk(k, carry):
            off = base + k * CHUNK
            pltpu.sync_copy(srcr.at[pl.ds(off, CHUNK)], src_v)
            pltpu.sync_copy(dstr.at[pl.ds(off, CHUNK)], dst_v)
            pltpu.sync_copy(ur.at[pl.ds(off, CHUNK)], u_v)
            pltpu.async_copy(table.at[src_v], rows_v, sem).wait()

            def group(g, c2):
                u16 = u_v[pl.ds(g * 16, 16)]
                u16 = jnp.clip(u16, 0.0, 1.0)
                for j in range(16):
                    e = g * 16 + j
                    msg_v[e, 0:16] = (rows_v[e, 0:16]
                                      + u16[j] * rows_v[e, 16:32])
                return c2
            lax.fori_loop(0, CHUNK // 16, group, 0)
            pltpu.sync_copy(msg_v, acc.at[dst_v], add=True)
            return carry
        lax.fori_loop(0, NCH, chunk, 0)

        plsc.subcore_barrier()
        pltpu.sync_copy(acc.at[pl.ds(s * ZR, ZR)],
                        out.at[c, pl.ds(s * ZR, ZR)])

    return edge_pass


_edge_pass32 = _make_edge_pass(32)
_edge_pass16 = _make_edge_pass(16)


def _l1_body(x_ref, w_ref, r_ref, pd_ref, xr_ref):
    x = x_ref[...]
    p0 = jnp.dot(x, w_ref[0], preferred_element_type=jnp.float32)
    p1 = jnp.dot(x, w_ref[1], preferred_element_type=jnp.float32)
    pd_ref[...] = jnp.concatenate([p0, p1 - p0], axis=1)
    xr_ref[...] = jnp.dot(x, r_ref[...], preferred_element_type=jnp.float32)


_l1 = pl.pallas_call(
    _l1_body,
    out_shape=(jax.ShapeDtypeStruct((N, 32), jnp.float32),
               jax.ShapeDtypeStruct((N, HID), jnp.float32)),
)


def _mid_body(parts_ref, xr_ref, b1_ref, w2_ref, r2_ref, b2_ref,
              qd_ref, hrb_ref, cinv_ref):
    p = parts_ref[...]
    acc = p[0, :N] + p[1, :N]
    cinv = 1.0 / jnp.maximum(acc[:, 16:17], 1.0)
    h = acc[:, 0:16] * cinv + xr_ref[...] + b1_ref[...]
    h = jnp.where(h > 0, h, jnp.exp(jnp.minimum(h, 0.0)) - 1.0)
    q0 = jnp.dot(h, w2_ref[0], preferred_element_type=jnp.float32)
    q1 = jnp.dot(h, w2_ref[1], preferred_element_type=jnp.float32)
    qd_ref[...] = jnp.concatenate([q0, q1 - q0], axis=1)
    hrb_ref[...] = (jnp.dot(h, r2_ref[...], preferred_element_type=jnp.float32)
                    + b2_ref[...])
    cinv_ref[...] = jnp.broadcast_to(cinv, (N, HID))


_mid = pl.pallas_call(
    _mid_body,
    out_shape=(jax.ShapeDtypeStruct((N, 32), jnp.float32),
               jax.ShapeDtypeStruct((N, HID), jnp.float32),
               jax.ShapeDtypeStruct((N, HID), jnp.float32)),
)


def _fin_body(parts_ref, cinv_ref, hrb_ref, o_ref):
    p = parts_ref[...]
    o_ref[...] = (p[0, :N] + p[1, :N]) * cinv_ref[...] + hrb_ref[...]


_fin = pl.pallas_call(
    _fin_body,
    out_shape=jax.ShapeDtypeStruct((N, HID), jnp.float32),
)


def kernel(x, edge_index, edge_attr, W1, root1, b1, W2, root2, b2):
    src = edge_index[0]
    dst = edge_index[1]
    u = edge_attr[:, 0]
    pad = EPAD - E
    src_p = jnp.concatenate([src, jnp.zeros((pad,), src.dtype)])
    dst_p = jnp.concatenate([dst, jnp.full((pad,), N, dst.dtype)])
    u_p = jnp.concatenate([u, jnp.zeros((pad,), u.dtype)])

    pd1, xr1 = _l1(x, W1, root1)
    parts1 = _edge_pass32(pd1, src_p, dst_p, u_p)
    qd2, hrb, cinv = _mid(parts1, xr1, b1.reshape(1, HID), W2, root2,
                          b2.reshape(1, HID))
    parts2 = _edge_pass16(qd2, src_p, dst_p, u_p)
    return _fin(parts2, cinv, hrb)
